# Initial kernel scaffold; baseline (speedup 1.0000x reference)
import jax
import jax.numpy as jnp
from jax.experimental import pallas as pl


def kernel(skewsym_params):
    raise NotImplementedError("write your pallas kernel here")


# speedup vs baseline: 297.9383x; 297.9383x over previous
import functools

import jax
import jax.numpy as jnp
import numpy as np
from jax import lax
from jax.experimental import pallas as pl
from jax.experimental.pallas import tpu as pltpu
from jax.experimental.pallas import tpu_sc as plsc

N = 4096
NPARAMS = N * (N - 1) // 2
B = 128
SW = B + 8
NB = N // B
NW = 32
LANES = 16

_pairs = [(bi, bj) for bi in range(NB) for bj in range(bi + 1, NB)]
NPAIR = len(_pairs)
TSLOTS = -(-NPAIR // NW)
PADP = NW * TSLOTS + LANES
_bi_tab = np.full((PADP,), 0, np.int32)
_bj_tab = np.full((PADP,), 1, np.int32)
for _p, (_a, _b) in enumerate(_pairs):
    _bi_tab[_p] = _a
    _bj_tab[_p] = _b


def _offset(i):
    return i * (N - 1) - ((i * (i - 1)) >> 1)


def _body(params_hbm, bi_hbm, bj_hbm, out_hbm, tabi, tabj, sblk, ublk, tblk, sem):
    cid = lax.axis_index("c")
    sid = lax.axis_index("s")
    wid = sid * 2 + cid

    pltpu.sync_copy(bi_hbm, tabi)
    pltpu.sync_copy(bj_hbm, tabj)

    iota = lax.iota(jnp.int32, LANES)

    def splat(x):
        return jnp.full((LANES,), 0, jnp.int32) + x

    def shift_of(s):
        return jnp.maximum(s & 7, s - (NPARAMS - SW))

    def drain_staged():
        def wait_one(r, carry):
            pltpu.make_async_copy(
                params_hbm.at[pl.ds(0, SW)], sblk.at[0], sem
            ).wait()
            return carry

        lax.fori_loop(0, B, wait_one, 0, unroll=4)

    def stage(start_of_row):
        def fire(r, carry):
            s = start_of_row(r)
            sa = pl.multiple_of(jnp.minimum(s - (s & 7), NPARAMS - SW), 8)
            pltpu.make_async_copy(
                params_hbm.at[pl.ds(sa, SW)], sblk.at[r], sem
            ).start()
            return carry

        lax.fori_loop(0, B, fire, 0, unroll=4)
        drain_staged()

    def do_pair(bi, bj):
        r0 = bi * B
        c0 = bj * B

        def start_of_row(r):
            i = r0 + r
            return _offset(i) + c0 - i - 1

        stage(start_of_row)

        def realign(r, carry):
            d = shift_of(start_of_row(r))
            for k in range(B // LANES):
                idx = d + k * LANES + iota
                v = plsc.load_gather(sblk, [splat(r), idx])
                ublk[r, pl.ds(k * LANES, LANES)] = v
            return carry

        lax.fori_loop(0, B, realign, 0)
        pltpu.sync_copy(ublk, out_hbm.at[pl.ds(r0, B), pl.ds(c0, B)])

        def col_body(c, carry):
            for k in range(B // LANES):
                rows = k * LANES + iota
                v = plsc.load_gather(ublk, [rows, splat(c)])
                tblk[c, pl.ds(k * LANES, LANES)] = -v
            return carry

        lax.fori_loop(0, B, col_body, 0)
        pltpu.sync_copy(tblk, out_hbm.at[pl.ds(c0, B), pl.ds(r0, B)])

    def slot(t, carry):
        p = wid + NW * t

        @pl.when(p < NPAIR)
        def _():
            bi = tabi[pl.ds(p, LANES)][0]
            bj = tabj[pl.ds(p, LANES)][0]
            do_pair(bi, bj)

        return carry

    lax.fori_loop(0, TSLOTS, slot, 0)

    r0 = wid * B

    def start_of_diag_row(r):
        return jnp.maximum(0, _offset(r0 + r) - r - 1)

    stage(start_of_diag_row)

    def drow(r, carry):
        i = r0 + r
        corr_r = jnp.where(i == 0, 1, 0).astype(jnp.int32)
        d_r = shift_of(start_of_diag_row(r))
        for k in range(B // LANES):
            jv = k * LANES + iota
            ucols = jnp.clip(jv - corr_r, 0, B - 1) + d_r
            vu = plsc.load_gather(sblk, [splat(r), ucols])
            gi = r0 + jv
            corr_j = jnp.where(gi == 0, 1, 0).astype(jnp.int32)
            sj = jnp.maximum(0, gi * (N - 1) - ((gi * (gi - 1)) >> 1) - jv - 1)
            d_j = shift_of(sj)
            lcols = jnp.clip(r - corr_j, 0, B - 1) + d_j
            vl = plsc.load_gather(sblk, [jv, lcols])
            zero = jnp.zeros((LANES,), jnp.float32)
            val = jnp.where(jv > r, vu, zero) + jnp.where(jv < r, -vl, zero)
            tblk[r, pl.ds(k * LANES, LANES)] = val
        return carry

    lax.fori_loop(0, B, drow, 0)
    pltpu.sync_copy(tblk, out_hbm.at[pl.ds(r0, B), pl.ds(r0, B)])


@jax.jit
def kernel(skewsym_params):
    mesh = plsc.VectorSubcoreMesh(core_axis_name="c", subcore_axis_name="s")
    f = pl.kernel(
        _body,
        out_type=jax.ShapeDtypeStruct((N, N), jnp.float32),
        mesh=mesh,
        compiler_params=pltpu.CompilerParams(
            use_tc_tiling_on_sc=False, needs_layout_passes=False
        ),
        scratch_types=[
            pltpu.VMEM((PADP,), jnp.int32),
            pltpu.VMEM((PADP,), jnp.int32),
            pltpu.VMEM((B, SW), jnp.float32),
            pltpu.VMEM((B, B), jnp.float32),
            pltpu.VMEM((B, B), jnp.float32),
            pltpu.SemaphoreType.DMA,
        ],
    )
    return f(skewsym_params, jnp.asarray(_bi_tab), jnp.asarray(_bj_tab))


# speedup vs baseline: 343.7127x; 1.1536x over previous
import functools

import jax
import jax.numpy as jnp
import numpy as np
from jax import lax
from jax.experimental import pallas as pl
from jax.experimental.pallas import tpu as pltpu
from jax.experimental.pallas import tpu_sc as plsc

N = 4096
NPARAMS = N * (N - 1) // 2
B = 128
SW = B + 8
NB = N // B
NW = 32
LANES = 16

_pairs = [(bi, bj) for bi in range(NB) for bj in range(bi + 1, NB)]
NPAIR = len(_pairs)
TSLOTS = -(-NPAIR // NW)
PADP = NW * TSLOTS + LANES
_bi_tab = np.full((PADP,), 0, np.int32)
_bj_tab = np.full((PADP,), 1, np.int32)
for _p, (_a, _b) in enumerate(_pairs):
    _bi_tab[_p] = _a
    _bj_tab[_p] = _b


def _offset(i):
    return i * (N - 1) - ((i * (i - 1)) >> 1)


def _body(params_hbm, bi_hbm, bj_hbm, out_hbm, tabi, tabj, sblk, ublk, tblk,
          dbuf, sem, sem_out):
    cid = lax.axis_index("c")
    sid = lax.axis_index("s")
    wid = sid * 2 + cid

    pltpu.sync_copy(bi_hbm, tabi)
    pltpu.sync_copy(bj_hbm, tabj)

    iota = lax.iota(jnp.int32, LANES)

    def splat(x):
        return jnp.full((LANES,), 0, jnp.int32) + x

    def shift_of(s):
        return jnp.maximum(s & 7, s - (NPARAMS - SW))

    def drain_staged():
        def wait_one(r, carry):
            pltpu.make_async_copy(
                params_hbm.at[pl.ds(0, SW)], sblk.at[0], sem
            ).wait()
            return carry

        lax.fori_loop(0, B, wait_one, 0, unroll=4)

    def stage(start_of_row):
        def fire(r, carry):
            s = start_of_row(r)
            sa = pl.multiple_of(jnp.minimum(s - (s & 7), NPARAMS - SW), 8)
            pltpu.make_async_copy(
                params_hbm.at[pl.ds(sa, SW)], sblk.at[r], sem
            ).start()
            return carry

        lax.fori_loop(0, B, fire, 0, unroll=4)
        drain_staged()

    def do_pair(bi, bj):
        r0 = pl.multiple_of(bi * B, B)
        c0 = pl.multiple_of(bj * B, B)

        def start_of_row(r):
            i = r0 + r
            return _offset(i) + c0 - i - 1

        stage(start_of_row)

        def realign(r, carry):
            d = shift_of(start_of_row(r))
            for k in range(B // LANES):
                ublk[r, pl.ds(k * LANES, LANES)] = sblk[
                    r, pl.ds(d + k * LANES, LANES)
                ]
            return carry

        lax.fori_loop(0, B, realign, 0, unroll=2)
        pltpu.make_async_copy(
            ublk, out_hbm.at[pl.ds(r0, B), pl.ds(c0, B)], sem_out
        ).start()

        for k in range(B // LANES):
            rows = k * LANES + iota
            iv = r0 + rows
            sv = iv * (N - 1) - ((iv * (iv - 1)) >> 1) + c0 - iv - 1
            dbuf[pl.ds(k * LANES, LANES)] = shift_of(sv)

        def col_body(c, carry):
            for k in range(B // LANES):
                rows = k * LANES + iota
                dv = dbuf[pl.ds(k * LANES, LANES)]
                v = plsc.load_gather(sblk, [rows, dv + c])
                tblk[c, pl.ds(k * LANES, LANES)] = -v
            return carry

        lax.fori_loop(0, B, col_body, 0, unroll=2)
        pltpu.make_async_copy(
            tblk, out_hbm.at[pl.ds(c0, B), pl.ds(r0, B)], sem_out
        ).start()

        pltpu.make_async_copy(
            out_hbm.at[pl.ds(0, B), pl.ds(0, B)], tblk, sem_out
        ).wait()
        pltpu.make_async_copy(
            out_hbm.at[pl.ds(0, B), pl.ds(0, B)], tblk, sem_out
        ).wait()

    def slot(t, carry):
        p = wid + NW * t

        @pl.when(p < NPAIR)
        def _():
            bi = tabi[pl.ds(p, LANES)][0]
            bj = tabj[pl.ds(p, LANES)][0]
            do_pair(bi, bj)

        return carry

    lax.fori_loop(0, TSLOTS, slot, 0)

    r0 = wid * B

    def start_of_diag_row(r):
        return jnp.maximum(0, _offset(r0 + r) - r - 1)

    stage(start_of_diag_row)

    def drow(r, carry):
        i = r0 + r
        corr_r = jnp.where(i == 0, 1, 0).astype(jnp.int32)
        d_r = shift_of(start_of_diag_row(r))
        for k in range(B // LANES):
            jv = k * LANES + iota
            ucols = jnp.clip(jv - corr_r, 0, B - 1) + d_r
            vu = plsc.load_gather(sblk, [splat(r), ucols])
            gi = r0 + jv
            corr_j = jnp.where(gi == 0, 1, 0).astype(jnp.int32)
            sj = jnp.maximum(0, gi * (N - 1) - ((gi * (gi - 1)) >> 1) - jv - 1)
            d_j = shift_of(sj)
            lcols = jnp.clip(r - corr_j, 0, B - 1) + d_j
            vl = plsc.load_gather(sblk, [jv, lcols])
            zero = jnp.zeros((LANES,), jnp.float32)
            val = jnp.where(jv > r, vu, zero) + jnp.where(jv < r, -vl, zero)
            tblk[r, pl.ds(k * LANES, LANES)] = val
        return carry

    lax.fori_loop(0, B, drow, 0)
    pltpu.sync_copy(tblk, out_hbm.at[pl.ds(r0, B), pl.ds(r0, B)])


@jax.jit
def kernel(skewsym_params):
    mesh = plsc.VectorSubcoreMesh(core_axis_name="c", subcore_axis_name="s")
    f = pl.kernel(
        _body,
        out_type=jax.ShapeDtypeStruct((N, N), jnp.float32),
        mesh=mesh,
        compiler_params=pltpu.CompilerParams(
            use_tc_tiling_on_sc=False, needs_layout_passes=False
        ),
        scratch_types=[
            pltpu.VMEM((PADP,), jnp.int32),
            pltpu.VMEM((PADP,), jnp.int32),
            pltpu.VMEM((B, SW), jnp.float32),
            pltpu.VMEM((B, B), jnp.float32),
            pltpu.VMEM((B, B), jnp.float32),
            pltpu.VMEM((B,), jnp.int32),
            pltpu.SemaphoreType.DMA,
            pltpu.SemaphoreType.DMA,
        ],
    )
    return f(skewsym_params, jnp.asarray(_bi_tab), jnp.asarray(_bj_tab))


# speedup vs baseline: 369.8860x; 1.0761x over previous
import functools

import jax
import jax.numpy as jnp
import numpy as np
from jax import lax
from jax.experimental import pallas as pl
from jax.experimental.pallas import tpu as pltpu
from jax.experimental.pallas import tpu_sc as plsc

N = 4096
NPARAMS = N * (N - 1) // 2
B = 128
SW = B + 8
NB = N // B
NW = 32
LANES = 16

_pairs = [(bi, bj) for bi in range(NB) for bj in range(bi + 1, NB)]
NPAIR = len(_pairs)
TSLOTS = -(-NPAIR // NW)
PADP = NW * TSLOTS + LANES
_bi_tab = np.full((PADP,), 0, np.int32)
_bj_tab = np.full((PADP,), 1, np.int32)
for _p, (_a, _b) in enumerate(_pairs):
    _bi_tab[_p] = _a
    _bj_tab[_p] = _b


def _offset(i):
    return i * (N - 1) - ((i * (i - 1)) >> 1)


def _body(params_hbm, bi_hbm, bj_hbm, out_hbm, tabi, tabj,
          sblk0, sblk1, ublk0, ublk1, tblk0, tblk1, dbuf,
          sem_in0, sem_in1, sem_out0, sem_out1):
    cid = lax.axis_index("c")
    sid = lax.axis_index("s")
    wid = sid * 2 + cid

    SB = (sblk0, sblk1)
    UB = (ublk0, ublk1)
    TB = (tblk0, tblk1)
    SI = (sem_in0, sem_in1)
    SO = (sem_out0, sem_out1)

    pltpu.sync_copy(bi_hbm, tabi)
    pltpu.sync_copy(bj_hbm, tabj)

    iota = lax.iota(jnp.int32, LANES)

    def shift_of(s):
        return jnp.maximum(s & 7, s - (NPARAMS - SW))

    def pair_of(p):
        bi = tabi[pl.ds(p, LANES)][0]
        bj = tabj[pl.ds(p, LANES)][0]
        return bi, bj

    def pair_start(bi, bj):
        def start_of_row(r):
            i = bi * B + r
            return _offset(i) + bj * B - i - 1

        return start_of_row

    def diag_start(r):
        return jnp.maximum(0, _offset(wid * B + r) - r - 1)

    def fire_rows(start_of_row, par):
        def fire(r, carry):
            s = start_of_row(r)
            sa = pl.multiple_of(jnp.minimum(s - (s & 7), NPARAMS - SW), 8)
            pltpu.make_async_copy(
                params_hbm.at[pl.ds(sa, SW)], SB[par].at[r], SI[par]
            ).start()
            return carry

        lax.fori_loop(0, B, fire, 0, unroll=4)

    def drain_in(par):
        pltpu.make_async_copy(
            out_hbm.at[pl.ds(0, B), pl.ds(0, SW)], SB[par], SI[par]
        ).wait()

    def drain_out(par):
        pltpu.make_async_copy(
            out_hbm.at[pl.ds(0, B), pl.ds(0, B)], TB[par], SO[par]
        ).wait()

    def fire_pair(jt, par):
        p = wid + NW * jt

        @pl.when(p < NPAIR)
        def _():
            bi, bj = pair_of(p)
            fire_rows(pair_start(bi, bj), par)

    def compute_pair(jt, tt, par):
        p = wid + NW * jt

        @pl.when(p < NPAIR)
        def _():
            drain_in(par)

            @pl.when(tt >= 1)
            def _():
                drain_out(par)
                drain_out(par)

            bi, bj = pair_of(p)
            r0 = pl.multiple_of(bi * B, B)
            c0 = pl.multiple_of(bj * B, B)
            start_of_row = pair_start(bi, bj)
            sblk, ublk, tblk = SB[par], UB[par], TB[par]

            def realign(r, carry):
                d = shift_of(start_of_row(r))
                for k in range(B // LANES):
                    ublk[r, pl.ds(k * LANES, LANES)] = sblk[
                        r, pl.ds(d + k * LANES, LANES)
                    ]
                return carry

            lax.fori_loop(0, B, realign, 0, unroll=2)
            pltpu.make_async_copy(
                ublk, out_hbm.at[pl.ds(r0, B), pl.ds(c0, B)], SO[par]
            ).start()

            for k in range(B // LANES):
                iv = r0 + k * LANES + iota
                sv = iv * (N - 1) - ((iv * (iv - 1)) >> 1) + c0 - iv - 1
                dbuf[pl.ds(k * LANES, LANES)] = shift_of(sv)

            def col_body(c, carry):
                for k in range(B // LANES):
                    rows = k * LANES + iota
                    dv = dbuf[pl.ds(k * LANES, LANES)]
                    v = plsc.load_gather(sblk, [rows, dv + c])
                    tblk[c, pl.ds(k * LANES, LANES)] = -v
                return carry

            lax.fori_loop(0, B, col_body, 0, unroll=2)
            pltpu.make_async_copy(
                tblk, out_hbm.at[pl.ds(c0, B), pl.ds(r0, B)], SO[par]
            ).start()

    fire_pair(0, 0)

    def loop_body(tt, carry):
        jt_a = 2 * tt
        fire_pair(jt_a + 1, 1)
        compute_pair(jt_a, tt, 0)

        nxt = jt_a + 2

        @pl.when(nxt == TSLOTS)
        def _():
            fire_rows(diag_start, 0)

        @pl.when(nxt < TSLOTS)
        def _():
            fire_pair(nxt, 0)

        compute_pair(jt_a + 1, tt, 1)
        return carry

    lax.fori_loop(0, TSLOTS // 2, loop_body, 0)

    r0 = wid * B
    drain_in(0)
    drain_out(0)
    drain_out(0)
    sblk, tblk = SB[0], TB[0]

    def drow(r, carry):
        i = r0 + r
        corr_r = jnp.where(i == 0, 1, 0).astype(jnp.int32)
        d_r = shift_of(diag_start(r))
        for k in range(B // LANES):
            jv = k * LANES + iota
            ucols = jnp.clip(jv - corr_r, 0, B - 1) + d_r
            vu = plsc.load_gather(
                sblk, [jnp.full((LANES,), 0, jnp.int32) + r, ucols]
            )
            gi = r0 + jv
            corr_j = jnp.where(gi == 0, 1, 0).astype(jnp.int32)
            sj = jnp.maximum(0, gi * (N - 1) - ((gi * (gi - 1)) >> 1) - jv - 1)
            d_j = shift_of(sj)
            lcols = jnp.clip(r - corr_j, 0, B - 1) + d_j
            vl = plsc.load_gather(sblk, [jv, lcols])
            zero = jnp.zeros((LANES,), jnp.float32)
            val = jnp.where(jv > r, vu, zero) + jnp.where(jv < r, -vl, zero)
            tblk[r, pl.ds(k * LANES, LANES)] = val
        return carry

    lax.fori_loop(0, B, drow, 0)
    pltpu.make_async_copy(
        tblk, out_hbm.at[pl.ds(r0, B), pl.ds(r0, B)], SO[0]
    ).start()

    drain_out(1)
    drain_out(1)
    drain_out(0)


@jax.jit
def kernel(skewsym_params):
    mesh = plsc.VectorSubcoreMesh(core_axis_name="c", subcore_axis_name="s")
    f = pl.kernel(
        _body,
        out_type=jax.ShapeDtypeStruct((N, N), jnp.float32),
        mesh=mesh,
        compiler_params=pltpu.CompilerParams(
            use_tc_tiling_on_sc=False, needs_layout_passes=False
        ),
        scratch_types=[
            pltpu.VMEM((PADP,), jnp.int32),
            pltpu.VMEM((PADP,), jnp.int32),
            pltpu.VMEM((B, SW), jnp.float32),
            pltpu.VMEM((B, SW), jnp.float32),
            pltpu.VMEM((B, B), jnp.float32),
            pltpu.VMEM((B, B), jnp.float32),
            pltpu.VMEM((B, B), jnp.float32),
            pltpu.VMEM((B, B), jnp.float32),
            pltpu.VMEM((B,), jnp.int32),
            pltpu.SemaphoreType.DMA,
            pltpu.SemaphoreType.DMA,
            pltpu.SemaphoreType.DMA,
            pltpu.SemaphoreType.DMA,
        ],
    )
    return f(skewsym_params, jnp.asarray(_bi_tab), jnp.asarray(_bj_tab))
